# baseline (speedup 1.0000x reference)
import functools

import jax
import jax.numpy as jnp
from jax.experimental import pallas as pl
from jax.experimental.pallas import tpu as pltpu


def _moe_body(x_ref, w1_ref, w2_ref, ids_ref, tw_ref, o_ref, acc_ref, *, n_exp, d_ff):
    e = pl.program_id(1)

    @pl.when(e == 0)
    def _init():
        acc_ref[...] = jnp.zeros_like(acc_ref)

    x = x_ref[...]
    h = jax.lax.dot_general(x, w1_ref[...], (((1,), (1,)), ((), ())),
                            preferred_element_type=jnp.float32)
    act = (jax.nn.silu(h[:, :d_ff]) * h[:, d_ff:]).astype(jnp.bfloat16)
    y = jax.lax.dot_general(act, w2_ref[...], (((1,), (1,)), ((), ())),
                            preferred_element_type=jnp.float32)
    ids = ids_ref[...]
    tw = tw_ref[...]
    gate = jnp.sum(jnp.where(ids == e, tw, 0.0), axis=1)
    acc_ref[...] += y * gate[:, None]

    @pl.when(e == n_exp - 1)
    def _flush():
        o_ref[...] = acc_ref[...]


def kernel(hidden_states, w1, w2, topk_weights, topk_ids):
    m, d_model = hidden_states.shape
    n_exp, two_n, _ = w1.shape
    d_ff = w2.shape[2]
    bm = 512

    x_bf = hidden_states.astype(jnp.bfloat16)
    w1_bf = w1.astype(jnp.bfloat16)
    w2_bf = w2.astype(jnp.bfloat16)

    body = functools.partial(_moe_body, n_exp=n_exp, d_ff=d_ff)
    return pl.pallas_call(
        body,
        grid=(m // bm, n_exp),
        in_specs=[
            pl.BlockSpec((bm, d_model), lambda i, e: (i, 0)),
            pl.BlockSpec((None, two_n, d_model), lambda i, e: (e, 0, 0)),
            pl.BlockSpec((None, d_model, d_ff), lambda i, e: (e, 0, 0)),
            pl.BlockSpec((bm, topk_ids.shape[1]), lambda i, e: (i, 0)),
            pl.BlockSpec((bm, topk_weights.shape[1]), lambda i, e: (i, 0)),
        ],
        out_specs=pl.BlockSpec((bm, d_model), lambda i, e: (i, 0)),
        out_shape=jax.ShapeDtypeStruct((m, d_model), jnp.float32),
        scratch_shapes=[pltpu.VMEM((bm, d_model), jnp.float32)],
        compiler_params=pltpu.CompilerParams(
            dimension_semantics=("parallel", "arbitrary")),
    )(x_bf, w1_bf, w2_bf, topk_ids, topk_weights)


# speedup vs baseline: 1.2447x; 1.2447x over previous
import functools

import jax
import jax.numpy as jnp
from jax.experimental import pallas as pl
from jax.experimental.pallas import tpu as pltpu


def _moe_body(x_ref, w1_ref, w2_ref, ids_ref, tw_ref, o_ref, acc_ref, *, n_exp, d_ff):
    e = pl.program_id(1)

    @pl.when(e == 0)
    def _init():
        acc_ref[...] = jnp.zeros_like(acc_ref)

    x = x_ref[...].astype(jnp.bfloat16)
    h = jax.lax.dot_general(x, w1_ref[...].astype(jnp.bfloat16), (((1,), (1,)), ((), ())),
                            preferred_element_type=jnp.float32)
    act = (jax.nn.silu(h[:, :d_ff]) * h[:, d_ff:]).astype(jnp.bfloat16)
    y = jax.lax.dot_general(act, w2_ref[...].astype(jnp.bfloat16), (((1,), (1,)), ((), ())),
                            preferred_element_type=jnp.float32)
    ids = ids_ref[...]
    tw = tw_ref[...]
    gate = jnp.sum(jnp.where(ids == e, tw, 0.0), axis=1)
    acc_ref[...] += y * gate[:, None]

    @pl.when(e == n_exp - 1)
    def _flush():
        o_ref[...] = acc_ref[...]


def kernel(hidden_states, w1, w2, topk_weights, topk_ids):
    m, d_model = hidden_states.shape
    n_exp, two_n, _ = w1.shape
    d_ff = w2.shape[2]
    bm = 512

    body = functools.partial(_moe_body, n_exp=n_exp, d_ff=d_ff)
    return pl.pallas_call(
        body,
        grid=(m // bm, n_exp),
        in_specs=[
            pl.BlockSpec((bm, d_model), lambda i, e: (i, 0)),
            pl.BlockSpec((None, two_n, d_model), lambda i, e: (e, 0, 0)),
            pl.BlockSpec((None, d_model, d_ff), lambda i, e: (e, 0, 0)),
            pl.BlockSpec((bm, topk_ids.shape[1]), lambda i, e: (i, 0)),
            pl.BlockSpec((bm, topk_weights.shape[1]), lambda i, e: (i, 0)),
        ],
        out_specs=pl.BlockSpec((bm, d_model), lambda i, e: (i, 0)),
        out_shape=jax.ShapeDtypeStruct((m, d_model), jnp.float32),
        scratch_shapes=[pltpu.VMEM((bm, d_model), jnp.float32)],
        compiler_params=pltpu.CompilerParams(
            dimension_semantics=("parallel", "arbitrary")),
    )(hidden_states, w1, w2, topk_ids, topk_weights)
